# baseline (speedup 1.0000x reference)
import functools

import jax
import jax.numpy as jnp
from jax.experimental import pallas as pl
from jax.experimental.pallas import tpu as pltpu
from jax.experimental.pallas import tpu_sc as plsc

N = 2048
D = 1024
FF = 1024
E = 8
K = 2
T = 128
_TSH = T.bit_length() - 1
G = (N * K) // T + (E - 1)
GP = 40
P = G * T
RB = 256

_NW = 32
_CH = N // _NW
_HH = _CH // 2


def _router_body(x_ref, gw_ref, logits_ref, m0_ref, m1_ref, w0_ref,
                 off_ref, teo_ref, counts_ref):
    b = pl.program_id(0)

    @pl.when(b == 0)
    def _():
        counts_ref[...] = jnp.zeros_like(counts_ref)

    x = x_ref[...]
    gw = gw_ref[...]
    logits = jax.lax.dot_general(x, gw, (((1,), (1,)), ((), ())),
                                 preferred_element_type=jnp.float32)
    logits_ref[...] = logits

    iota = jax.lax.broadcasted_iota(jnp.int32, (RB, E), 1)
    m0 = jnp.max(logits, axis=1, keepdims=True)
    e0 = jnp.min(jnp.where(logits == m0, iota, E), axis=1, keepdims=True)
    l1 = jnp.where(iota == e0, -jnp.inf, logits)
    m1 = jnp.max(l1, axis=1, keepdims=True)
    e1 = jnp.min(jnp.where(l1 == m1, iota, E), axis=1, keepdims=True)
    w0_ref[...] = 1.0 / (1.0 + jnp.exp(m1 - m0))

    oh0 = (iota == e0).astype(jnp.float32)
    oh1 = (iota == e1).astype(jnp.float32)
    comb = oh0 + oh1
    r = jax.lax.broadcasted_iota(jnp.int32, (RB, RB), 0)
    c = jax.lax.broadcasted_iota(jnp.int32, (RB, RB), 1)
    tril = (r > c).astype(jnp.float32)
    prefix = jax.lax.dot_general(tril, comb, (((1,), (0,)), ((), ())),
                                 preferred_element_type=jnp.float32)
    prefix = prefix + counts_ref[0:1, :]
    r0 = jnp.sum(prefix * oh0, axis=1, keepdims=True).astype(jnp.int32)
    r1 = jnp.sum(prefix * oh1, axis=1, keepdims=True).astype(jnp.int32)
    m0_ref[...] = (e0 << 12) + r0
    m1_ref[...] = (e1 << 12) + r1
    colsum = jnp.sum(comb, axis=0, keepdims=True)
    newcnt = counts_ref[0:1, :] + colsum
    counts_ref[...] = jnp.broadcast_to(newcnt, (8, E))

    cpad = jnp.floor((newcnt + (T - 1)) * (1.0 / T)) * T
    eiota_r = jax.lax.broadcasted_iota(jnp.int32, (E, E), 0)
    eiota_c = jax.lax.broadcasted_iota(jnp.int32, (E, E), 1)
    sut = (eiota_r < eiota_c).astype(jnp.float32)
    off_row = jax.lax.dot_general(cpad, sut, (((1,), (0,)), ((), ())),
                                  preferred_element_type=jnp.float32)
    lane16 = jax.lax.broadcasted_iota(jnp.int32, (8, 16), 1)
    off16 = jnp.pad(off_row, ((0, 0), (0, 8)))
    off_ref[...] = jnp.where(lane16 < E,
                             jnp.broadcast_to(off16, (8, 16)),
                             0).astype(jnp.int32)
    ct = (off_row + cpad) * (1.0 / T)
    t_iota = jax.lax.broadcasted_iota(jnp.int32, (GP, E), 0).astype(jnp.float32)
    tev = jnp.sum((t_iota >= ct).astype(jnp.float32), axis=1, keepdims=True)
    tev = jnp.minimum(tev, float(E - 1))
    nt_b = jnp.broadcast_to(ct[:, E - 1:E], (GP, 1))
    row_iota = jax.lax.broadcasted_iota(jnp.int32, (GP, 1), 0)
    teo_ref[...] = jnp.where(row_iota == GP - 1, nt_b, tev).astype(jnp.int32)


_router = pl.pallas_call(
    _router_body,
    grid=(N // RB,),
    in_specs=[
        pl.BlockSpec((RB, D), lambda b: (b, 0)),
        pl.BlockSpec((E, D), lambda b: (0, 0)),
    ],
    out_specs=[
        pl.BlockSpec((RB, E), lambda b: (b, 0)),
        pl.BlockSpec((RB, 1), lambda b: (b, 0)),
        pl.BlockSpec((RB, 1), lambda b: (b, 0)),
        pl.BlockSpec((RB, 1), lambda b: (b, 0)),
        pl.BlockSpec((8, 16), lambda b: (0, 0)),
        pl.BlockSpec((GP, 1), lambda b: (0, 0)),
    ],
    out_shape=[
        jax.ShapeDtypeStruct((N, E), jnp.float32),
        jax.ShapeDtypeStruct((N, 1), jnp.int32),
        jax.ShapeDtypeStruct((N, 1), jnp.int32),
        jax.ShapeDtypeStruct((N, 1), jnp.float32),
        jax.ShapeDtypeStruct((8, 16), jnp.int32),
        jax.ShapeDtypeStruct((GP, 1), jnp.int32),
    ],
    scratch_shapes=[pltpu.VMEM((8, E), jnp.float32)],
)


_sc_mesh = plsc.VectorSubcoreMesh(core_axis_name="c", subcore_axis_name="s",
                                  num_cores=2, num_subcores=16)


@functools.partial(
    pl.kernel,
    out_type=[
        jax.ShapeDtypeStruct((P, D), jnp.float32),
        jax.ShapeDtypeStruct((N,), jnp.int32),
        jax.ShapeDtypeStruct((N,), jnp.int32),
    ],
    mesh=_sc_mesh,
    compiler_params=pltpu.CompilerParams(needs_layout_passes=False),
    scratch_types=[
        pltpu.VMEM((16,), jnp.int32),
        pltpu.VMEM((_CH,), jnp.int32),
        pltpu.VMEM((_CH,), jnp.int32),
        pltpu.VMEM((_CH,), jnp.int32),
        pltpu.VMEM((_CH,), jnp.int32),
        pltpu.VMEM((_CH, D), jnp.float32),
        pltpu.SemaphoreType.DMA,
        pltpu.SemaphoreType.DMA,
        pltpu.SemaphoreType.DMA,
    ],
)
def _dispatch(x_hbm, m0_hbm, m1_hbm, off_hbm,
              xs_hbm, pos0_hbm, pos1_hbm,
              off_v, m0_v, m1_v, p0_v, p1_v, x_v,
              sem0, sem1, semx):
    wid = jax.lax.axis_index("s") * 2 + jax.lax.axis_index("c")
    base = wid * _CH
    cx = pltpu.async_copy(x_hbm.at[pl.ds(base, _CH)], x_v, semx)
    pltpu.sync_copy(off_hbm, off_v)
    pltpu.sync_copy(m0_hbm.at[pl.ds(base, _CH)], m0_v)
    pltpu.sync_copy(m1_hbm.at[pl.ds(base, _CH)], m1_v)
    for j in range(_CH // 16):
        sl = pl.ds(j * 16, 16)
        mm0 = m0_v[sl]
        mm1 = m1_v[sl]
        p0_v[sl] = plsc.load_gather(off_v, [mm0 >> 12]) + (mm0 & 0xFFF)
        p1_v[sl] = plsc.load_gather(off_v, [mm1 >> 12]) + (mm1 & 0xFFF)
    cx.wait()
    cp0 = pltpu.async_copy(x_v, xs_hbm.at[p0_v], sem0)
    cp1 = pltpu.async_copy(x_v, xs_hbm.at[p1_v], sem1)
    pltpu.sync_copy(p0_v, pos0_hbm.at[pl.ds(base, _CH)])
    pltpu.sync_copy(p1_v, pos1_hbm.at[pl.ds(base, _CH)])
    cp0.wait()
    cp1.wait()


def _mm_body(teo_ref, xs_ref, w1_ref, w2_ref, ym_ref):
    t = pl.program_id(0)

    @pl.when(t < teo_ref[GP - 1, 0])
    def _():
        xb = xs_ref[...]
        h = jnp.dot(xb, w1_ref[0], preferred_element_type=jnp.float32)
        h = 0.5 * h * (1.0 + jax.lax.erf(h * 0.7071067811865476))
        ym_ref[...] = jnp.dot(h, w2_ref[0], preferred_element_type=jnp.float32)


_grouped_mm = pl.pallas_call(
    _mm_body,
    grid_spec=pltpu.PrefetchScalarGridSpec(
        num_scalar_prefetch=1,
        grid=(G,),
        in_specs=[
            pl.BlockSpec((T, D), lambda t, teo: (jnp.minimum(t, teo[GP - 1, 0] - 1), 0)),
            pl.BlockSpec((1, D, FF), lambda t, teo: (teo[t, 0], 0, 0)),
            pl.BlockSpec((1, FF, D), lambda t, teo: (teo[t, 0], 0, 0)),
        ],
        out_specs=pl.BlockSpec((T, D), lambda t, teo: (t, 0)),
    ),
    out_shape=jax.ShapeDtypeStruct((P, D), jnp.float32),
)


_QH = 16
_NQ = _CH // _QH


@functools.partial(
    pl.kernel,
    out_type=jax.ShapeDtypeStruct((N, D), jnp.float32),
    mesh=_sc_mesh,
    compiler_params=pltpu.CompilerParams(needs_layout_passes=False),
    scratch_types=[
        pltpu.VMEM((_CH,), jnp.int32),
        pltpu.VMEM((_CH,), jnp.int32),
        pltpu.VMEM((_CH,), jnp.float32),
        pltpu.VMEM((_QH, D), jnp.float32),
        pltpu.VMEM((_QH, D), jnp.float32),
        pltpu.VMEM((_QH, D), jnp.float32),
        pltpu.VMEM((_QH, D), jnp.float32),
        pltpu.SemaphoreType.DMA,
        pltpu.SemaphoreType.DMA,
        pltpu.SemaphoreType.DMA,
        pltpu.SemaphoreType.DMA,
    ],
)
def _combine(ym_hbm, pos0_hbm, pos1_hbm, w0_hbm, y_hbm,
             p0_v, p1_v, w0_v, a0_v, a1_v, b0_v, b1_v,
             sa0, sa1, sb0, sb1):
    wid = jax.lax.axis_index("s") * 2 + jax.lax.axis_index("c")
    base = wid * _CH
    pltpu.sync_copy(w0_hbm.at[pl.ds(base, _CH)], w0_v)
    pltpu.sync_copy(pos0_hbm.at[pl.ds(base, _CH)], p0_v)
    pltpu.sync_copy(pos1_hbm.at[pl.ds(base, _CH)], p1_v)
    abufs = (a0_v, a1_v)
    bbufs = (b0_v, b1_v)
    asems = (sa0, sa1)
    bsems = (sb0, sb1)

    def gather(q):
        cur = q % 2
        i0 = p0_v[pl.ds(q * _QH, _QH)]
        i1 = p1_v[pl.ds(q * _QH, _QH)]
        ca = pltpu.async_copy(ym_hbm.at[i0], abufs[cur], asems[cur])
        cb = pltpu.async_copy(ym_hbm.at[i1], bbufs[cur], bsems[cur])
        return ca, cb

    pending = gather(0)
    for q in range(_NQ):
        cur = q % 2
        pending[0].wait()
        pending[1].wait()
        if q + 1 < _NQ:
            pending = gather(q + 1)
        a_v = abufs[cur]
        b_v = bbufs[cur]

        def row_body(r, carry):
            ir = jnp.zeros((16,), jnp.int32) + (q * _QH + r)
            wa = plsc.load_gather(w0_v, [ir])
            wb = 1.0 - wa
            for cc in range(D // 16):
                sl = pl.ds(cc * 16, 16)
                a_v[r, sl] = a_v[r, sl] * wa + b_v[r, sl] * wb
            return carry

        jax.lax.fori_loop(0, _QH, row_body, 0)
        pltpu.sync_copy(a_v, y_hbm.at[pl.ds(base + q * _QH, _QH)])


@jax.jit
def kernel(x, gate_w, w1, w2):
    logits, meta0, meta1, w0, off16, teo = _router(x, gate_w)
    xs, pos0, pos1 = _dispatch(x, meta0[:, 0], meta1[:, 0], off16[0])
    ym = _grouped_mm(teo, xs, w1, w2)
    y = _combine(ym, pos0, pos1, w0[:, 0])
    return y, logits


# speedup vs baseline: 1.0873x; 1.0873x over previous
import functools

import jax
import jax.numpy as jnp
from jax.experimental import pallas as pl
from jax.experimental.pallas import tpu as pltpu
from jax.experimental.pallas import tpu_sc as plsc

N = 2048
D = 1024
FF = 1024
E = 8
K = 2
T = 256
_TSH = T.bit_length() - 1
G = (N * K) // T + (E - 1)
GP = 32
P = G * T
RB = 256

_NW = 32
_CH = N // _NW
_HH = _CH // 2


def _router_body(x_ref, gw_ref, logits_ref, mc_ref, w0_ref,
                 off_ref, teo_ref, counts_ref):
    b = pl.program_id(0)

    @pl.when(b == 0)
    def _():
        counts_ref[...] = jnp.zeros_like(counts_ref)

    x = x_ref[...]
    gw = gw_ref[...]
    logits = jax.lax.dot_general(x, gw, (((1,), (1,)), ((), ())),
                                 preferred_element_type=jnp.float32)
    logits_ref[...] = logits

    iota = jax.lax.broadcasted_iota(jnp.int32, (RB, E), 1)
    m0 = jnp.max(logits, axis=1, keepdims=True)
    e0 = jnp.min(jnp.where(logits == m0, iota, E), axis=1, keepdims=True)
    l1 = jnp.where(iota == e0, -jnp.inf, logits)
    m1 = jnp.max(l1, axis=1, keepdims=True)
    e1 = jnp.min(jnp.where(l1 == m1, iota, E), axis=1, keepdims=True)
    w0_ref[...] = 1.0 / (1.0 + jnp.exp(m1 - m0))

    oh0 = (iota == e0).astype(jnp.float32)
    oh1 = (iota == e1).astype(jnp.float32)
    comb = oh0 + oh1
    r = jax.lax.broadcasted_iota(jnp.int32, (RB, RB), 0)
    c = jax.lax.broadcasted_iota(jnp.int32, (RB, RB), 1)
    tril = (r > c).astype(jnp.float32)
    prefix = jax.lax.dot_general(tril, comb, (((1,), (0,)), ((), ())),
                                 preferred_element_type=jnp.float32)
    prefix = prefix + counts_ref[0:1, :]
    r0 = jnp.sum(prefix * oh0, axis=1, keepdims=True).astype(jnp.int32)
    r1 = jnp.sum(prefix * oh1, axis=1, keepdims=True).astype(jnp.int32)
    mc_ref[...] = ((((e0 << 12) + r0) << 15)
                   + (e1 << 12) + r1)
    colsum = jnp.sum(comb, axis=0, keepdims=True)
    newcnt = counts_ref[0:1, :] + colsum
    counts_ref[...] = jnp.broadcast_to(newcnt, (8, E))

    cpad = jnp.floor((newcnt + (T - 1)) * (1.0 / T)) * T
    eiota_r = jax.lax.broadcasted_iota(jnp.int32, (E, E), 0)
    eiota_c = jax.lax.broadcasted_iota(jnp.int32, (E, E), 1)
    sut = (eiota_r < eiota_c).astype(jnp.float32)
    off_row = jax.lax.dot_general(cpad, sut, (((1,), (0,)), ((), ())),
                                  preferred_element_type=jnp.float32)
    lane16 = jax.lax.broadcasted_iota(jnp.int32, (8, 16), 1)
    off16 = jnp.pad(off_row, ((0, 0), (0, 8)))
    off_ref[...] = jnp.where(lane16 < E,
                             jnp.broadcast_to(off16, (8, 16)),
                             0).astype(jnp.int32)
    ct = (off_row + cpad) * (1.0 / T)
    t_iota = jax.lax.broadcasted_iota(jnp.int32, (GP, E), 0).astype(jnp.float32)
    tev = jnp.sum((t_iota >= ct).astype(jnp.float32), axis=1, keepdims=True)
    tev = jnp.minimum(tev, float(E - 1))
    nt_b = jnp.broadcast_to(ct[:, E - 1:E], (GP, 1))
    row_iota = jax.lax.broadcasted_iota(jnp.int32, (GP, 1), 0)
    teo_ref[...] = jnp.where(row_iota == GP - 1, nt_b, tev).astype(jnp.int32)


_router = pl.pallas_call(
    _router_body,
    grid=(N // RB,),
    in_specs=[
        pl.BlockSpec((RB, D), lambda b: (b, 0)),
        pl.BlockSpec((E, D), lambda b: (0, 0)),
    ],
    out_specs=[
        pl.BlockSpec((RB, E), lambda b: (b, 0)),
        pl.BlockSpec((RB, 1), lambda b: (b, 0)),
        pl.BlockSpec((RB, 1), lambda b: (b, 0)),
        pl.BlockSpec((8, 16), lambda b: (0, 0)),
        pl.BlockSpec((GP, 1), lambda b: (0, 0)),
    ],
    out_shape=[
        jax.ShapeDtypeStruct((N, E), jnp.float32),
        jax.ShapeDtypeStruct((N, 1), jnp.int32),
        jax.ShapeDtypeStruct((N, 1), jnp.float32),
        jax.ShapeDtypeStruct((8, 16), jnp.int32),
        jax.ShapeDtypeStruct((GP, 1), jnp.int32),
    ],
    scratch_shapes=[pltpu.VMEM((8, E), jnp.float32)],
)


_sc_mesh = plsc.VectorSubcoreMesh(core_axis_name="c", subcore_axis_name="s",
                                  num_cores=2, num_subcores=16)


@functools.partial(
    pl.kernel,
    out_type=[
        jax.ShapeDtypeStruct((P, D), jnp.float32),
        jax.ShapeDtypeStruct((N,), jnp.int32),
        jax.ShapeDtypeStruct((N,), jnp.int32),
    ],
    mesh=_sc_mesh,
    compiler_params=pltpu.CompilerParams(needs_layout_passes=False),
    scratch_types=[
        pltpu.VMEM((16,), jnp.int32),
        pltpu.VMEM((_CH,), jnp.int32),
        pltpu.VMEM((_CH,), jnp.int32),
        pltpu.VMEM((_CH,), jnp.int32),
        pltpu.VMEM((_CH, D), jnp.float32),
        pltpu.SemaphoreType.DMA,
        pltpu.SemaphoreType.DMA,
        pltpu.SemaphoreType.DMA,
    ],
)
def _dispatch(x_hbm, mc_hbm, off_hbm,
              xs_hbm, pos0_hbm, pos1_hbm,
              off_v, mc_v, p0_v, p1_v, x_v,
              sem0, sem1, semx):
    wid = jax.lax.axis_index("s") * 2 + jax.lax.axis_index("c")
    base = wid * _CH
    cx = pltpu.async_copy(x_hbm.at[pl.ds(base, _CH)], x_v, semx)
    pltpu.sync_copy(off_hbm, off_v)
    pltpu.sync_copy(mc_hbm.at[pl.ds(base, _CH)], mc_v)
    for j in range(_CH // 16):
        sl = pl.ds(j * 16, 16)
        mm0 = mc_v[sl] >> 15
        mm1 = mc_v[sl] & 0x7FFF
        p0_v[sl] = plsc.load_gather(off_v, [mm0 >> 12]) + (mm0 & 0xFFF)
        p1_v[sl] = plsc.load_gather(off_v, [mm1 >> 12]) + (mm1 & 0xFFF)
    cx.wait()
    cp0 = pltpu.async_copy(x_v, xs_hbm.at[p0_v], sem0)
    cp1 = pltpu.async_copy(x_v, xs_hbm.at[p1_v], sem1)
    pltpu.sync_copy(p0_v, pos0_hbm.at[pl.ds(base, _CH)])
    pltpu.sync_copy(p1_v, pos1_hbm.at[pl.ds(base, _CH)])
    cp0.wait()
    cp1.wait()


def _mm_body(teo_ref, xs_ref, w1_ref, w2_ref, ym_ref):
    t = pl.program_id(0)

    @pl.when(t < teo_ref[GP - 1, 0])
    def _():
        xb = xs_ref[...]
        h = jnp.dot(xb, w1_ref[0], preferred_element_type=jnp.float32)
        h = 0.5 * h * (1.0 + jax.lax.erf(h * 0.7071067811865476))
        ym_ref[...] = jnp.dot(h, w2_ref[0], preferred_element_type=jnp.float32)


_grouped_mm = pl.pallas_call(
    _mm_body,
    grid_spec=pltpu.PrefetchScalarGridSpec(
        num_scalar_prefetch=1,
        grid=(G,),
        in_specs=[
            pl.BlockSpec((T, D),
                         lambda t, teo: (jnp.minimum(t, teo[GP - 1, 0] - 1), 0)),
            pl.BlockSpec((1, D, FF), lambda t, teo: (teo[t, 0], 0, 0)),
            pl.BlockSpec((1, FF, D), lambda t, teo: (teo[t, 0], 0, 0)),
        ],
        out_specs=pl.BlockSpec((T, D), lambda t, teo: (t, 0)),
    ),
    out_shape=jax.ShapeDtypeStruct((P, D), jnp.float32),
)


_QH = 16
_NQ = _CH // _QH


@functools.partial(
    pl.kernel,
    out_type=jax.ShapeDtypeStruct((N, D), jnp.float32),
    mesh=_sc_mesh,
    compiler_params=pltpu.CompilerParams(needs_layout_passes=False),
    scratch_types=[
        pltpu.VMEM((_CH,), jnp.int32),
        pltpu.VMEM((_CH,), jnp.int32),
        pltpu.VMEM((_CH,), jnp.float32),
        pltpu.VMEM((_QH, D), jnp.float32),
        pltpu.VMEM((_QH, D), jnp.float32),
        pltpu.VMEM((_QH, D), jnp.float32),
        pltpu.VMEM((_QH, D), jnp.float32),
        pltpu.SemaphoreType.DMA,
        pltpu.SemaphoreType.DMA,
        pltpu.SemaphoreType.DMA,
        pltpu.SemaphoreType.DMA,
    ],
)
def _combine(ym_hbm, pos0_hbm, pos1_hbm, w0_hbm, y_hbm,
             p0_v, p1_v, w0_v, a0_v, a1_v, b0_v, b1_v,
             sa0, sa1, sb0, sb1):
    wid = jax.lax.axis_index("s") * 2 + jax.lax.axis_index("c")
    base = wid * _CH
    pltpu.sync_copy(w0_hbm.at[pl.ds(base, _CH)], w0_v)
    pltpu.sync_copy(pos0_hbm.at[pl.ds(base, _CH)], p0_v)
    pltpu.sync_copy(pos1_hbm.at[pl.ds(base, _CH)], p1_v)
    abufs = (a0_v, a1_v)
    bbufs = (b0_v, b1_v)
    asems = (sa0, sa1)
    bsems = (sb0, sb1)

    def gather(q):
        cur = q % 2
        i0 = p0_v[pl.ds(q * _QH, _QH)]
        i1 = p1_v[pl.ds(q * _QH, _QH)]
        ca = pltpu.async_copy(ym_hbm.at[i0], abufs[cur], asems[cur])
        cb = pltpu.async_copy(ym_hbm.at[i1], bbufs[cur], bsems[cur])
        return ca, cb

    pending = gather(0)
    for q in range(_NQ):
        cur = q % 2
        pending[0].wait()
        pending[1].wait()
        if q + 1 < _NQ:
            pending = gather(q + 1)
        a_v = abufs[cur]
        b_v = bbufs[cur]

        def row_body(r, carry):
            ir = jnp.zeros((16,), jnp.int32) + (q * _QH + r)
            wa = plsc.load_gather(w0_v, [ir])
            wb = 1.0 - wa
            for cc in range(D // 16):
                sl = pl.ds(cc * 16, 16)
                a_v[r, sl] = a_v[r, sl] * wa + b_v[r, sl] * wb
            return carry

        jax.lax.fori_loop(0, _QH, row_body, 0)
        pltpu.sync_copy(a_v, y_hbm.at[pl.ds(base + q * _QH, _QH)])


@jax.jit
def kernel(x, gate_w, w1, w2):
    logits, metac, w0, off16, teo = _router(x, gate_w)
    xs, pos0, pos1 = _dispatch(x, metac[:, 0], off16[0])
    ym = _grouped_mm(teo, xs, w1, w2)
    y = _combine(ym, pos0, pos1, w0[:, 0])
    return y, logits


# speedup vs baseline: 1.1170x; 1.0273x over previous
import functools

import jax
import jax.numpy as jnp
from jax.experimental import pallas as pl
from jax.experimental.pallas import tpu as pltpu
from jax.experimental.pallas import tpu_sc as plsc

N = 2048
D = 1024
FF = 1024
E = 8
K = 2
T = 256
_TSH = T.bit_length() - 1
G = (N * K) // T + (E - 1)
GP = 32
P = G * T
RB = 512

_NW = 32
_CH = N // _NW
_HH = _CH // 2


def _router_body(x_ref, gw_ref, logits_ref, mc_ref, w0_ref,
                 off_ref, teo_ref, counts_ref):
    b = pl.program_id(0)

    @pl.when(b == 0)
    def _():
        counts_ref[...] = jnp.zeros_like(counts_ref)

    x = x_ref[...]
    gw = gw_ref[...]
    logits = jax.lax.dot_general(x, gw, (((1,), (1,)), ((), ())),
                                 preferred_element_type=jnp.float32)
    logits_ref[...] = logits

    iota = jax.lax.broadcasted_iota(jnp.int32, (RB, E), 1)
    m0 = jnp.max(logits, axis=1, keepdims=True)
    e0 = jnp.min(jnp.where(logits == m0, iota, E), axis=1, keepdims=True)
    l1 = jnp.where(iota == e0, -jnp.inf, logits)
    m1 = jnp.max(l1, axis=1, keepdims=True)
    e1 = jnp.min(jnp.where(l1 == m1, iota, E), axis=1, keepdims=True)
    w0_ref[...] = 1.0 / (1.0 + jnp.exp(m1 - m0))

    oh0 = (iota == e0).astype(jnp.float32)
    oh1 = (iota == e1).astype(jnp.float32)
    comb = oh0 + oh1
    r = jax.lax.broadcasted_iota(jnp.int32, (RB, RB), 0)
    c = jax.lax.broadcasted_iota(jnp.int32, (RB, RB), 1)
    tril = (r > c).astype(jnp.float32)
    prefix = jax.lax.dot_general(tril, comb, (((1,), (0,)), ((), ())),
                                 preferred_element_type=jnp.float32)
    prefix = prefix + counts_ref[0:1, :]
    r0 = jnp.sum(prefix * oh0, axis=1, keepdims=True).astype(jnp.int32)
    r1 = jnp.sum(prefix * oh1, axis=1, keepdims=True).astype(jnp.int32)
    mc_ref[...] = ((((e0 << 12) + r0) << 15)
                   + (e1 << 12) + r1)
    colsum = jnp.sum(comb, axis=0, keepdims=True)
    newcnt = counts_ref[0:1, :] + colsum
    counts_ref[...] = jnp.broadcast_to(newcnt, (8, E))

    cpad = jnp.floor((newcnt + (T - 1)) * (1.0 / T)) * T
    eiota_r = jax.lax.broadcasted_iota(jnp.int32, (E, E), 0)
    eiota_c = jax.lax.broadcasted_iota(jnp.int32, (E, E), 1)
    sut = (eiota_r < eiota_c).astype(jnp.float32)
    off_row = jax.lax.dot_general(cpad, sut, (((1,), (0,)), ((), ())),
                                  preferred_element_type=jnp.float32)
    lane16 = jax.lax.broadcasted_iota(jnp.int32, (8, 16), 1)
    off16 = jnp.pad(off_row, ((0, 0), (0, 8)))
    off_ref[...] = jnp.where(lane16 < E,
                             jnp.broadcast_to(off16, (8, 16)),
                             0).astype(jnp.int32)
    ct = (off_row + cpad) * (1.0 / T)
    t_iota = jax.lax.broadcasted_iota(jnp.int32, (GP, E), 0).astype(jnp.float32)
    tev = jnp.sum((t_iota >= ct).astype(jnp.float32), axis=1, keepdims=True)
    tev = jnp.minimum(tev, float(E - 1))
    nt_b = jnp.broadcast_to(ct[:, E - 1:E], (GP, 1))
    row_iota = jax.lax.broadcasted_iota(jnp.int32, (GP, 1), 0)
    teo_ref[...] = jnp.where(row_iota == GP - 1, nt_b, tev).astype(jnp.int32)


_router = pl.pallas_call(
    _router_body,
    grid=(N // RB,),
    in_specs=[
        pl.BlockSpec((RB, D), lambda b: (b, 0)),
        pl.BlockSpec((E, D), lambda b: (0, 0)),
    ],
    out_specs=[
        pl.BlockSpec((RB, E), lambda b: (b, 0)),
        pl.BlockSpec((RB, 1), lambda b: (b, 0)),
        pl.BlockSpec((RB, 1), lambda b: (b, 0)),
        pl.BlockSpec((8, 16), lambda b: (0, 0)),
        pl.BlockSpec((GP, 1), lambda b: (0, 0)),
    ],
    out_shape=[
        jax.ShapeDtypeStruct((N, E), jnp.float32),
        jax.ShapeDtypeStruct((N, 1), jnp.int32),
        jax.ShapeDtypeStruct((N, 1), jnp.float32),
        jax.ShapeDtypeStruct((8, 16), jnp.int32),
        jax.ShapeDtypeStruct((GP, 1), jnp.int32),
    ],
    scratch_shapes=[pltpu.VMEM((8, E), jnp.float32)],
)


_sc_mesh = plsc.VectorSubcoreMesh(core_axis_name="c", subcore_axis_name="s",
                                  num_cores=2, num_subcores=16)


@functools.partial(
    pl.kernel,
    out_type=[
        jax.ShapeDtypeStruct((P, D), jnp.float32),
        jax.ShapeDtypeStruct((N,), jnp.int32),
        jax.ShapeDtypeStruct((N,), jnp.int32),
    ],
    mesh=_sc_mesh,
    compiler_params=pltpu.CompilerParams(needs_layout_passes=False),
    scratch_types=[
        pltpu.VMEM((16,), jnp.int32),
        pltpu.VMEM((_CH,), jnp.int32),
        pltpu.VMEM((_CH,), jnp.int32),
        pltpu.VMEM((_CH,), jnp.int32),
        pltpu.VMEM((_CH, D), jnp.float32),
        pltpu.SemaphoreType.DMA,
        pltpu.SemaphoreType.DMA,
        pltpu.SemaphoreType.DMA,
    ],
)
def _dispatch(x_hbm, mc_hbm, off_hbm,
              xs_hbm, pos0_hbm, pos1_hbm,
              off_v, mc_v, p0_v, p1_v, x_v,
              sem0, sem1, semx):
    wid = jax.lax.axis_index("s") * 2 + jax.lax.axis_index("c")
    base = wid * _CH
    cx = pltpu.async_copy(x_hbm.at[pl.ds(base, _CH)], x_v, semx)
    pltpu.sync_copy(off_hbm, off_v)
    pltpu.sync_copy(mc_hbm.at[pl.ds(base, _CH)], mc_v)
    for j in range(_CH // 16):
        sl = pl.ds(j * 16, 16)
        mm0 = mc_v[sl] >> 15
        mm1 = mc_v[sl] & 0x7FFF
        p0_v[sl] = plsc.load_gather(off_v, [mm0 >> 12]) + (mm0 & 0xFFF)
        p1_v[sl] = plsc.load_gather(off_v, [mm1 >> 12]) + (mm1 & 0xFFF)
    cx.wait()
    cp0 = pltpu.async_copy(x_v, xs_hbm.at[p0_v], sem0)
    cp1 = pltpu.async_copy(x_v, xs_hbm.at[p1_v], sem1)
    pltpu.sync_copy(p0_v, pos0_hbm.at[pl.ds(base, _CH)])
    pltpu.sync_copy(p1_v, pos1_hbm.at[pl.ds(base, _CH)])
    cp0.wait()
    cp1.wait()


def _mm_body(teo_ref, xs_ref, w1_ref, w2_ref, ym_ref):
    t = pl.program_id(0)

    @pl.when(t < teo_ref[GP - 1, 0])
    def _():
        xb = xs_ref[...]
        h = jnp.dot(xb, w1_ref[0], preferred_element_type=jnp.float32)
        h = 0.5 * h * (1.0 + jax.lax.erf(h * 0.7071067811865476))
        ym_ref[...] = jnp.dot(h, w2_ref[0], preferred_element_type=jnp.float32)


_grouped_mm = pl.pallas_call(
    _mm_body,
    grid_spec=pltpu.PrefetchScalarGridSpec(
        num_scalar_prefetch=1,
        grid=(G,),
        in_specs=[
            pl.BlockSpec((T, D),
                         lambda t, teo: (jnp.minimum(t, teo[GP - 1, 0] - 1), 0)),
            pl.BlockSpec((1, D, FF), lambda t, teo: (teo[t, 0], 0, 0)),
            pl.BlockSpec((1, FF, D), lambda t, teo: (teo[t, 0], 0, 0)),
        ],
        out_specs=pl.BlockSpec((T, D), lambda t, teo: (t, 0)),
    ),
    out_shape=jax.ShapeDtypeStruct((P, D), jnp.float32),
)


_QH = 16
_NQ = _CH // _QH


@functools.partial(
    pl.kernel,
    out_type=jax.ShapeDtypeStruct((N, D), jnp.float32),
    mesh=_sc_mesh,
    compiler_params=pltpu.CompilerParams(needs_layout_passes=False),
    scratch_types=[
        pltpu.VMEM((_CH,), jnp.int32),
        pltpu.VMEM((_CH,), jnp.int32),
        pltpu.VMEM((_CH,), jnp.float32),
        pltpu.VMEM((_QH, D), jnp.float32),
        pltpu.VMEM((_QH, D), jnp.float32),
        pltpu.VMEM((_QH, D), jnp.float32),
        pltpu.VMEM((_QH, D), jnp.float32),
        pltpu.SemaphoreType.DMA,
        pltpu.SemaphoreType.DMA,
        pltpu.SemaphoreType.DMA,
        pltpu.SemaphoreType.DMA,
    ],
)
def _combine(ym_hbm, pos0_hbm, pos1_hbm, w0_hbm, y_hbm,
             p0_v, p1_v, w0_v, a0_v, a1_v, b0_v, b1_v,
             sa0, sa1, sb0, sb1):
    wid = jax.lax.axis_index("s") * 2 + jax.lax.axis_index("c")
    base = wid * _CH
    pltpu.sync_copy(w0_hbm.at[pl.ds(base, _CH)], w0_v)
    pltpu.sync_copy(pos0_hbm.at[pl.ds(base, _CH)], p0_v)
    pltpu.sync_copy(pos1_hbm.at[pl.ds(base, _CH)], p1_v)
    abufs = (a0_v, a1_v)
    bbufs = (b0_v, b1_v)
    asems = (sa0, sa1)
    bsems = (sb0, sb1)

    def gather(q):
        cur = q % 2
        i0 = p0_v[pl.ds(q * _QH, _QH)]
        i1 = p1_v[pl.ds(q * _QH, _QH)]
        ca = pltpu.async_copy(ym_hbm.at[i0], abufs[cur], asems[cur])
        cb = pltpu.async_copy(ym_hbm.at[i1], bbufs[cur], bsems[cur])
        return ca, cb

    pending = gather(0)
    for q in range(_NQ):
        cur = q % 2
        pending[0].wait()
        pending[1].wait()
        if q + 1 < _NQ:
            pending = gather(q + 1)
        a_v = abufs[cur]
        b_v = bbufs[cur]

        def row_body(r, carry):
            ir = jnp.zeros((16,), jnp.int32) + (q * _QH + r)
            wa = plsc.load_gather(w0_v, [ir])
            wb = 1.0 - wa
            for cc in range(D // 16):
                sl = pl.ds(cc * 16, 16)
                a_v[r, sl] = a_v[r, sl] * wa + b_v[r, sl] * wb
            return carry

        jax.lax.fori_loop(0, _QH, row_body, 0)
        pltpu.sync_copy(a_v, y_hbm.at[pl.ds(base + q * _QH, _QH)])


@jax.jit
def kernel(x, gate_w, w1, w2):
    logits, metac, w0, off16, teo = _router(x, gate_w)
    xs, pos0, pos1 = _dispatch(x, metac[:, 0], off16[0])
    ym = _grouped_mm(teo, xs, w1, w2)
    y = _combine(ym, pos0, pos1, w0[:, 0])
    return y, logits
